# Initial kernel scaffold; baseline (speedup 1.0000x reference)
import jax
import jax.numpy as jnp
from jax.experimental import pallas as pl


def kernel(gt, t_score, s_score):
    raise NotImplementedError("write your pallas kernel here")


# speedup vs baseline: 66.8529x; 66.8529x over previous
import jax
import jax.numpy as jnp
from jax import lax
from jax.experimental import pallas as pl
from jax.experimental.pallas import tpu as pltpu

_N_ROWS = 128
_N_COLS = 100000
_BLOCK_W = 2048
_N_BLOCKS = (_N_COLS + _BLOCK_W - 1) // _BLOCK_W


def _loss_body(t_ref, s_ref, out_ref, mt, zt, sa, ms, zs):
    k = pl.program_id(0)

    @pl.when(k == 0)
    def _init():
        mt[...] = jnp.full((_N_ROWS, 1), -jnp.inf, jnp.float32)
        ms[...] = jnp.full((_N_ROWS, 1), -jnp.inf, jnp.float32)
        zt[...] = jnp.zeros((_N_ROWS, 1), jnp.float32)
        zs[...] = jnp.zeros((_N_ROWS, 1), jnp.float32)
        sa[...] = jnp.zeros((_N_ROWS, 1), jnp.float32)

    cols = k * _BLOCK_W + lax.broadcasted_iota(jnp.int32, (1, _BLOCK_W), 1)
    valid = cols < _N_COLS
    t = jnp.where(valid, t_ref[...], -jnp.inf)
    s_raw = s_ref[...]
    s_lse = jnp.where(valid, s_raw, -jnp.inf)
    s_prod = jnp.where(valid, s_raw, 0.0)

    m_new = jnp.maximum(mt[...], jnp.max(t, axis=1, keepdims=True))
    alpha = jnp.exp(mt[...] - m_new)
    et = jnp.exp(t - m_new)
    zt[...] = zt[...] * alpha + jnp.sum(et, axis=1, keepdims=True)
    sa[...] = sa[...] * alpha + jnp.sum(et * s_prod, axis=1, keepdims=True)
    mt[...] = m_new

    m_new_s = jnp.maximum(ms[...], jnp.max(s_lse, axis=1, keepdims=True))
    beta = jnp.exp(ms[...] - m_new_s)
    zs[...] = zs[...] * beta + jnp.sum(jnp.exp(s_lse - m_new_s), axis=1, keepdims=True)
    ms[...] = m_new_s

    @pl.when(k == _N_BLOCKS - 1)
    def _finalize():
        per_row = sa[...] / zt[...] - ms[...] - jnp.log(zs[...])
        out_ref[...] = -jnp.mean(per_row).reshape(1, 1)


def kernel(gt, t_score, s_score):
    del gt
    out = pl.pallas_call(
        _loss_body,
        grid=(_N_BLOCKS,),
        in_specs=[
            pl.BlockSpec((_N_ROWS, _BLOCK_W), lambda k: (0, k)),
            pl.BlockSpec((_N_ROWS, _BLOCK_W), lambda k: (0, k)),
        ],
        out_specs=pl.BlockSpec((1, 1), lambda k: (0, 0)),
        out_shape=jax.ShapeDtypeStruct((1, 1), jnp.float32),
        scratch_shapes=[pltpu.VMEM((_N_ROWS, 1), jnp.float32) for _ in range(5)],
    )(t_score, s_score)
    return out[0, 0]


# speedup vs baseline: 67.7209x; 1.0130x over previous
import jax
import jax.numpy as jnp
from jax import lax
from jax.experimental import pallas as pl
from jax.experimental.pallas import tpu as pltpu

_N_ROWS = 128
_N_COLS = 100000
_BLOCK_W = 2048
_N_BLOCKS = (_N_COLS + _BLOCK_W - 1) // _BLOCK_W


def _accumulate(t, s_lse, s_prod, mt, zt, sa, ms, zs):
    m_new = jnp.maximum(mt[...], jnp.max(t, axis=1, keepdims=True))
    alpha = jnp.exp(mt[...] - m_new)
    et = jnp.exp(t - m_new)
    zt[...] = zt[...] * alpha + jnp.sum(et, axis=1, keepdims=True)
    sa[...] = sa[...] * alpha + jnp.sum(et * s_prod, axis=1, keepdims=True)
    mt[...] = m_new

    m_new_s = jnp.maximum(ms[...], jnp.max(s_lse, axis=1, keepdims=True))
    beta = jnp.exp(ms[...] - m_new_s)
    zs[...] = zs[...] * beta + jnp.sum(jnp.exp(s_lse - m_new_s), axis=1, keepdims=True)
    ms[...] = m_new_s


def _loss_body(t_ref, s_ref, out_ref, mt, zt, sa, ms, zs):
    k = pl.program_id(0)

    @pl.when(k == 0)
    def _init():
        mt[...] = jnp.full((_N_ROWS, 1), -jnp.inf, jnp.float32)
        ms[...] = jnp.full((_N_ROWS, 1), -jnp.inf, jnp.float32)
        zt[...] = jnp.zeros((_N_ROWS, 1), jnp.float32)
        zs[...] = jnp.zeros((_N_ROWS, 1), jnp.float32)
        sa[...] = jnp.zeros((_N_ROWS, 1), jnp.float32)

    @pl.when(k < _N_BLOCKS - 1)
    def _full_block():
        _accumulate(t_ref[...], s_ref[...], s_ref[...], mt, zt, sa, ms, zs)

    @pl.when(k == _N_BLOCKS - 1)
    def _tail_block():
        cols = k * _BLOCK_W + lax.broadcasted_iota(jnp.int32, (1, _BLOCK_W), 1)
        valid = cols < _N_COLS
        t = jnp.where(valid, t_ref[...], -jnp.inf)
        s_raw = s_ref[...]
        _accumulate(t, jnp.where(valid, s_raw, -jnp.inf),
                    jnp.where(valid, s_raw, 0.0), mt, zt, sa, ms, zs)

        per_row = sa[...] / zt[...] - ms[...] - jnp.log(zs[...])
        out_ref[...] = -jnp.mean(per_row).reshape(1, 1)


def kernel(gt, t_score, s_score):
    del gt
    out = pl.pallas_call(
        _loss_body,
        grid=(_N_BLOCKS,),
        in_specs=[
            pl.BlockSpec((_N_ROWS, _BLOCK_W), lambda k: (0, k)),
            pl.BlockSpec((_N_ROWS, _BLOCK_W), lambda k: (0, k)),
        ],
        out_specs=pl.BlockSpec((1, 1), lambda k: (0, 0)),
        out_shape=jax.ShapeDtypeStruct((1, 1), jnp.float32),
        scratch_shapes=[pltpu.VMEM((_N_ROWS, 1), jnp.float32) for _ in range(5)],
    )(t_score, s_score)
    return out[0, 0]
